# baseline (speedup 1.0000x reference)
import functools

import jax
import jax.numpy as jnp
from jax import lax
from jax.experimental import pallas as pl
from jax.experimental.pallas import tpu as pltpu
from jax.experimental.pallas import tpu_sc as plsc

NUM_CLASSES = 1000
HIDDEN = 1152
BATCH = 16384

_INFO = plsc.get_sparse_core_info()
NC = _INFO.num_cores
NS = _INFO.num_subcores
NW = NC * NS
B_PER_W = BATCH // NW


def _embed_body(table_hbm, labels_hbm, out_hbm, tbl_sh, idx_v, sem):
    sid = lax.axis_index("s")
    wid = sid * NC + lax.axis_index("c")
    base = wid * B_PER_W

    @pl.when(sid < 15)
    def _():
        pltpu.sync_copy(table_hbm.at[pl.ds(sid * (64 * HIDDEN), 64 * HIDDEN)],
                        tbl_sh.at[pl.ds(sid * (64 * HIDDEN), 64 * HIDDEN)])

    @pl.when(sid == 15)
    def _():
        pltpu.sync_copy(table_hbm.at[pl.ds(960 * HIDDEN, 40 * HIDDEN)],
                        tbl_sh.at[pl.ds(960 * HIDDEN, 40 * HIDDEN)])

    pltpu.sync_copy(labels_hbm.at[pl.ds(base, B_PER_W)], idx_v)
    plsc.subcore_barrier()

    cps = []
    for g in range(B_PER_W // 16):
        vec = idx_v[pl.ds(g * 16, 16)]
        for k in range(16):
            i = g * 16 + k
            src = pl.multiple_of(vec[k] * HIDDEN, HIDDEN)
            dst = pl.multiple_of((base + i) * HIDDEN, HIDDEN)
            cps.append(pltpu.async_copy(
                tbl_sh.at[pl.ds(src, HIDDEN)],
                out_hbm.at[pl.ds(dst, HIDDEN)], sem))
    for cp in cps:
        cp.wait()


@jax.jit
def _embed(labels, table_flat):
    mesh = plsc.VectorSubcoreMesh(core_axis_name="c", subcore_axis_name="s")
    f = pl.kernel(
        _embed_body,
        out_type=jax.ShapeDtypeStruct((BATCH * HIDDEN,), jnp.float32),
        mesh=mesh,
        compiler_params=pltpu.CompilerParams(
            disable_bounds_checks=True,
            disable_semaphore_checks=True,
        ),
        scratch_types=[
            pltpu.VMEM_SHARED((NUM_CLASSES * HIDDEN,), jnp.float32),
            pltpu.VMEM((B_PER_W,), jnp.int32),
            pltpu.SemaphoreType.DMA,
        ],
    )
    return f(table_flat, labels).reshape(BATCH, HIDDEN)


def kernel(labels, embedding_table):
    return _embed(labels.astype(jnp.int32), embedding_table.reshape(-1))
